# baseline (speedup 1.0000x reference)
import functools

import jax
import jax.numpy as jnp
from jax import lax
from jax.experimental import pallas as pl
from jax.experimental.pallas import tpu as pltpu
from jax.experimental.pallas import tpu_sc as plsc

NG = 4096
NE_PAD = 128
NC, NS, L = 2, 16, 16
NW = NC * NS
BLK = 2048
ATOT_PAD = 102400
ROW = 4104
GPW = NG // NW
NB = 48


@functools.cache
def _build():
  mesh = plsc.VectorSubcoreMesh(
      core_axis_name="c", subcore_axis_name="s",
      num_cores=NC, num_subcores=NS)

  @functools.partial(
      pl.kernel,
      out_type=jax.ShapeDtypeStruct((NG,), jnp.float32),
      mesh=mesh,
      compiler_params=pltpu.CompilerParams(needs_layout_passes=False),
      scratch_types=[
          pltpu.VMEM((BLK,), jnp.int32),
          pltpu.VMEM((BLK,), jnp.int32),
          pltpu.VMEM((NE_PAD,), jnp.float32),
          pltpu.VMEM((NB,), jnp.int32),
          pltpu.VMEM((ROW,), jnp.float32),
          pltpu.VMEM((GPW,), jnp.float32),
          pltpu.VMEM((GPW,), jnp.int32),
          pltpu.VMEM((L,), jnp.float32),
          pltpu.VMEM((L,), jnp.float32),
          pltpu.VMEM((GPW,), jnp.float32),
      ],
  )
  def _fused(z_hbm, img_hbm, ae_hbm, bounds_hbm, energy_hbm, natoms_hbm,
             scale_hbm, shift_hbm, out_hbm,
             z_v, g_v, ae_v, b_v, acc_v, en_v, na_v, sc_v, sh_v, res_v):
    c = lax.axis_index("c")
    s = lax.axis_index("s")
    w = s * NC + c
    g0 = w * GPW
    iota = lax.iota(jnp.int32, L)

    pltpu.sync_copy(bounds_hbm, b_v)
    pltpu.sync_copy(ae_hbm, ae_v)
    pltpu.sync_copy(energy_hbm.at[pl.ds(g0, GPW)], en_v)
    pltpu.sync_copy(natoms_hbm.at[pl.ds(g0, GPW)], na_v)
    pltpu.sync_copy(scale_hbm, sc_v)
    pltpu.sync_copy(shift_hbm, sh_v)

    be = plsc.load_gather(b_v, [w + jnp.minimum(iota, 1)])
    s_at = jnp.sum(jnp.where(iota == 0, be, 0))
    e_at = jnp.sum(jnp.where(iota == 1, be, 0))
    s8 = jnp.bitwise_and(s_at, -8)
    nblk = (e_at - s8 + (BLK - 1)) // BLK

    zeros = jnp.zeros((L,), jnp.float32)
    for k in range(GPW // L):
        plsc.store_scatter(acc_v, [g0 + k * L + iota], zeros)

    def blk_body(k, carry):
        off = pl.multiple_of(s8 + k * BLK, 8)
        pltpu.sync_copy(z_hbm.at[pl.ds(off, BLK)], z_v)
        pltpu.sync_copy(img_hbm.at[pl.ds(off, BLK)], g_v)
        for i in range(BLK // L):
            sl = pl.ds(i * L, L)
            vals = plsc.load_gather(ae_v, [z_v[sl]])
            plsc.addupdate_scatter(acc_v, [g_v[sl]], vals)
        return carry

    lax.fori_loop(0, nblk, blk_body, 0)

    scale = sc_v[pl.ds(0, L)]
    shift = sh_v[pl.ds(0, L)]
    for k in range(GPW // L):
        sl = pl.ds(k * L, L)
        e0 = plsc.load_gather(acc_v, [g0 + k * L + iota])
        res_v[sl] = en_v[sl] * scale + na_v[sl].astype(jnp.float32) * shift + e0
    pltpu.sync_copy(res_v, out_hbm.at[pl.ds(g0, GPW)])

  return _fused


def kernel(energy, n_atoms, Z, image_idx, scale_by, shift_by, atomic_energies):
    n_atoms_total = Z.shape[0]
    pad = ATOT_PAD - n_atoms_total
    img32 = image_idx.astype(jnp.int32)
    z_pad = jnp.concatenate(
        [Z.astype(jnp.int32), jnp.zeros((pad,), jnp.int32)])
    img_pad = jnp.concatenate([img32, jnp.full((pad,), NG, jnp.int32)])
    bounds = jnp.zeros((NB,), jnp.int32).at[:NW + 1].set(
        jnp.searchsorted(img32, jnp.arange(NW + 1, dtype=jnp.int32) * GPW,
                         side="left").astype(jnp.int32))
    ae = jnp.zeros((NE_PAD,), jnp.float32).at[: atomic_energies.shape[0]].set(
        atomic_energies)
    scale = jnp.broadcast_to(scale_by.astype(jnp.float32), (L,))
    shift = jnp.broadcast_to(shift_by.astype(jnp.float32), (L,))
    return _build()(z_pad, img_pad, ae, bounds, energy, n_atoms, scale, shift)


# speedup vs baseline: 1.5237x; 1.5237x over previous
import functools

import jax
import jax.numpy as jnp
from jax import lax
from jax.experimental import pallas as pl
from jax.experimental.pallas import tpu as pltpu
from jax.experimental.pallas import tpu_sc as plsc

NG = 4096
NA = 100000
NE = 119
NC, NS, L = 2, 16, 16
ROW = 4104
GPC = NG // NC
GPW = GPC // NS
APT = 6272
APT_LAST = NA - (NS - 1) * APT
UNROLL = 8


@functools.cache
def _build():
  mesh = plsc.VectorSubcoreMesh(
      core_axis_name="c", subcore_axis_name="s",
      num_cores=NC, num_subcores=NS)

  @functools.partial(
      pl.kernel,
      out_type=jax.ShapeDtypeStruct((NG,), jnp.float32),
      mesh=mesh,
      compiler_params=pltpu.CompilerParams(needs_layout_passes=False),
      scratch_types=[
          pltpu.VMEM((APT,), jnp.int32),
          pltpu.VMEM((APT,), jnp.int32),
          pltpu.VMEM((NE,), jnp.float32),
          pltpu.VMEM((ROW,), jnp.float32),
          pltpu.VMEM((GPW,), jnp.float32),
          pltpu.VMEM((GPW,), jnp.float32),
          pltpu.VMEM((GPW,), jnp.int32),
          pltpu.VMEM((1,), jnp.float32),
          pltpu.VMEM((1,), jnp.float32),
          pltpu.VMEM((GPW,), jnp.float32),
          pltpu.VMEM_SHARED((NS, ROW), jnp.float32),
      ],
  )
  def _fused(energy_hbm, natoms_hbm, z_hbm, img_hbm, scale_hbm, shift_hbm,
             ae_hbm, zrow_hbm, out_hbm,
             z_v, g_v, ae_v, acc_v, tmp_v, en_v, na_v, sc_v, sh_v, res_v,
             rows_sh):
    c = lax.axis_index("c")
    s = lax.axis_index("s")
    base = s * APT

    pltpu.sync_copy(ae_hbm, ae_v)
    pltpu.sync_copy(zrow_hbm, acc_v)

    @pl.when(s < NS - 1)
    def _():
        pltpu.sync_copy(z_hbm.at[pl.ds(base, APT)], z_v)
        pltpu.sync_copy(img_hbm.at[pl.ds(base, APT)], g_v)

    @pl.when(s == NS - 1)
    def _():
        pltpu.sync_copy(z_hbm.at[pl.ds(base, APT_LAST)],
                        z_v.at[pl.ds(0, APT_LAST)])
        pltpu.sync_copy(img_hbm.at[pl.ds(base, APT_LAST)],
                        g_v.at[pl.ds(0, APT_LAST)])

    def vec(i):
        sl = pl.ds(i * L, L)
        vals = plsc.load_gather(ae_v, [z_v[sl]])
        plsc.addupdate_scatter(acc_v, [g_v[sl]], vals)

    for i in range(APT_LAST // L):
        vec(i)

    @pl.when(s < NS - 1)
    def _():
        for i in range(APT_LAST // L, APT // L):
            vec(i)

    pltpu.sync_copy(acc_v, rows_sh.at[s])
    plsc.subcore_barrier()

    g0 = c * GPC + s * GPW
    pltpu.sync_copy(energy_hbm.at[pl.ds(g0, GPW)], en_v)
    pltpu.sync_copy(natoms_hbm.at[pl.ds(g0, GPW)], na_v)
    pltpu.sync_copy(scale_hbm, sc_v)
    pltpu.sync_copy(shift_hbm, sh_v)

    acc = [None] * (GPW // L)
    for r in range(NS):
        pltpu.sync_copy(rows_sh.at[r, pl.ds(g0, GPW)], tmp_v)
        for k in range(GPW // L):
            sl = pl.ds(k * L, L)
            v = tmp_v[sl]
            acc[k] = v if acc[k] is None else acc[k] + v

    zero16 = jnp.zeros((L,), jnp.int32)
    scale = plsc.load_gather(sc_v, [zero16])
    shift = plsc.load_gather(sh_v, [zero16])
    for k in range(GPW // L):
        sl = pl.ds(k * L, L)
        res_v[sl] = (en_v[sl] * scale
                     + na_v[sl].astype(jnp.float32) * shift + acc[k])
    pltpu.sync_copy(res_v, out_hbm.at[pl.ds(g0, GPW)])

  return _fused


def kernel(energy, n_atoms, Z, image_idx, scale_by, shift_by, atomic_energies):
    zrow = jnp.zeros((ROW,), jnp.float32)
    return _build()(
        energy, n_atoms.astype(jnp.int32), Z.astype(jnp.int32),
        image_idx.astype(jnp.int32), scale_by.astype(jnp.float32),
        shift_by.astype(jnp.float32), atomic_energies.astype(jnp.float32),
        zrow)


# speedup vs baseline: 1.8168x; 1.1924x over previous
import functools

import jax
import jax.numpy as jnp
from jax import lax
from jax.experimental import pallas as pl
from jax.experimental.pallas import tpu as pltpu
from jax.experimental.pallas import tpu_sc as plsc

NG = 4096
NA = 100000
NE = 119
NS, L = 16, 16
ROW = 4104
GPT = NG // NS
APT = 6272
APT_LAST = NA - (NS - 1) * APT


@functools.cache
def _build():
  mesh = plsc.VectorSubcoreMesh(
      core_axis_name="c", subcore_axis_name="s",
      num_cores=1, num_subcores=NS)

  @functools.partial(
      pl.kernel,
      out_type=jax.ShapeDtypeStruct((NG,), jnp.float32),
      mesh=mesh,
      compiler_params=pltpu.CompilerParams(needs_layout_passes=False),
      scratch_types=[
          pltpu.VMEM((APT,), jnp.int32),
          pltpu.VMEM((APT,), jnp.int32),
          pltpu.VMEM((NE,), jnp.float32),
          pltpu.VMEM((ROW,), jnp.float32),
          pltpu.VMEM((NS, GPT), jnp.float32),
          pltpu.VMEM((GPT,), jnp.float32),
          pltpu.VMEM((GPT,), jnp.int32),
          pltpu.VMEM((1,), jnp.float32),
          pltpu.VMEM((1,), jnp.float32),
          pltpu.VMEM((GPT,), jnp.float32),
          pltpu.VMEM_SHARED((NS, ROW), jnp.float32),
          pltpu.SemaphoreType.DMA,
          pltpu.SemaphoreType.DMA,
      ],
  )
  def _fused(energy_hbm, natoms_hbm, z_hbm, img_hbm, scale_hbm, shift_hbm,
             ae_hbm, zrow_hbm, out_hbm,
             z_v, g_v, ae_v, acc_v, cmb_v, en_v, na_v, sc_v, sh_v, res_v,
             rows_sh, semA, semB):
    s = lax.axis_index("s")
    base = s * APT
    g0 = s * GPT

    cp_ae = pltpu.async_copy(ae_hbm, ae_v, semA)
    cp_zero = pltpu.async_copy(zrow_hbm, acc_v, semA)
    cp_en = pltpu.async_copy(energy_hbm.at[pl.ds(g0, GPT)], en_v, semB)
    cp_na = pltpu.async_copy(natoms_hbm.at[pl.ds(g0, GPT)], na_v, semB)
    cp_sc = pltpu.async_copy(scale_hbm, sc_v, semB)
    cp_sh = pltpu.async_copy(shift_hbm, sh_v, semB)

    @pl.when(s < NS - 1)
    def _():
        pltpu.async_copy(z_hbm.at[pl.ds(base, APT)], z_v, semA).wait()
        pltpu.async_copy(img_hbm.at[pl.ds(base, APT)], g_v, semA).wait()

    @pl.when(s == NS - 1)
    def _():
        pltpu.async_copy(z_hbm.at[pl.ds(base, APT_LAST)],
                         z_v.at[pl.ds(0, APT_LAST)], semA).wait()
        pltpu.async_copy(img_hbm.at[pl.ds(base, APT_LAST)],
                         g_v.at[pl.ds(0, APT_LAST)], semA).wait()

    cp_ae.wait()
    cp_zero.wait()

    def vec(i):
        sl = pl.ds(i * L, L)
        vals = plsc.load_gather(ae_v, [z_v[sl]])
        plsc.addupdate_scatter(acc_v, [g_v[sl]], vals)

    for i in range(APT_LAST // L):
        vec(i)

    @pl.when(s < NS - 1)
    def _():
        for i in range(APT_LAST // L, APT // L):
            vec(i)

    pltpu.sync_copy(acc_v, rows_sh.at[s])
    plsc.subcore_barrier()
    pltpu.sync_copy(rows_sh.at[:, pl.ds(g0, GPT)], cmb_v)

    cp_en.wait()
    cp_na.wait()
    cp_sc.wait()
    cp_sh.wait()
    zero16 = jnp.zeros((L,), jnp.int32)
    scale = plsc.load_gather(sc_v, [zero16])
    shift = plsc.load_gather(sh_v, [zero16])
    for k in range(GPT // L):
        sl = pl.ds(k * L, L)
        acc = cmb_v[0, sl]
        for r in range(1, NS):
            acc = acc + cmb_v[r, sl]
        res_v[sl] = (en_v[sl] * scale
                     + na_v[sl].astype(jnp.float32) * shift + acc)
    pltpu.sync_copy(res_v, out_hbm.at[pl.ds(g0, GPT)])

  return _fused


def kernel(energy, n_atoms, Z, image_idx, scale_by, shift_by, atomic_energies):
    zrow = jnp.zeros((ROW,), jnp.float32)
    return _build()(
        energy, n_atoms.astype(jnp.int32), Z.astype(jnp.int32),
        image_idx.astype(jnp.int32), scale_by.astype(jnp.float32),
        shift_by.astype(jnp.float32), atomic_energies.astype(jnp.float32),
        zrow)


# speedup vs baseline: 1.9556x; 1.0764x over previous
import functools

import jax
import jax.numpy as jnp
from jax import lax
from jax.experimental import pallas as pl
from jax.experimental.pallas import tpu as pltpu
from jax.experimental.pallas import tpu_sc as plsc

NG = 4096
NA = 100000
NE = 119
NS, L = 16, 16
ROW = 4104
GPT = NG // NS
APT = 6272
APT_LAST = NA - (NS - 1) * APT


@functools.cache
def _build():
  mesh = plsc.VectorSubcoreMesh(
      core_axis_name="c", subcore_axis_name="s",
      num_cores=1, num_subcores=NS)

  @functools.partial(
      pl.kernel,
      out_type=jax.ShapeDtypeStruct((NG,), jnp.float32),
      mesh=mesh,
      compiler_params=pltpu.CompilerParams(needs_layout_passes=False),
      scratch_types=[
          pltpu.VMEM((APT,), jnp.int32),
          pltpu.VMEM((APT,), jnp.int32),
          pltpu.VMEM((NE,), jnp.float32),
          pltpu.VMEM((ROW,), jnp.float32),
          pltpu.VMEM((NS, GPT), jnp.float32),
          pltpu.VMEM((GPT,), jnp.float32),
          pltpu.VMEM((GPT,), jnp.int32),
          pltpu.VMEM((1,), jnp.float32),
          pltpu.VMEM((1,), jnp.float32),
          pltpu.VMEM((GPT,), jnp.float32),
          pltpu.VMEM_SHARED((NS, ROW), jnp.float32),
          pltpu.SemaphoreType.DMA,
          pltpu.SemaphoreType.DMA,
      ],
  )
  def _fused(energy_hbm, natoms_hbm, z_hbm, img_hbm, scale_hbm, shift_hbm,
             ae_hbm, zrow_hbm, out_hbm,
             z_v, g_v, ae_v, acc_v, cmb_v, en_v, na_v, sc_v, sh_v, res_v,
             rows_sh, semA, semB):
    s = lax.axis_index("s")
    base = s * APT
    g0 = s * GPT

    cp_ae = pltpu.async_copy(ae_hbm, ae_v, semA)
    cp_zero = pltpu.async_copy(zrow_hbm, acc_v, semA)
    cp_en = pltpu.async_copy(energy_hbm.at[pl.ds(g0, GPT)], en_v, semB)
    cp_na = pltpu.async_copy(natoms_hbm.at[pl.ds(g0, GPT)], na_v, semB)
    cp_sc = pltpu.async_copy(scale_hbm, sc_v, semB)
    cp_sh = pltpu.async_copy(shift_hbm, sh_v, semB)

    @pl.when(s < NS - 1)
    def _():
        pltpu.async_copy(z_hbm.at[pl.ds(base, APT)], z_v, semA).wait()
        pltpu.async_copy(img_hbm.at[pl.ds(base, APT)], g_v, semA).wait()

    @pl.when(s == NS - 1)
    def _():
        pltpu.async_copy(z_hbm.at[pl.ds(base, APT_LAST)],
                         z_v.at[pl.ds(0, APT_LAST)], semA).wait()
        pltpu.async_copy(img_hbm.at[pl.ds(base, APT_LAST)],
                         g_v.at[pl.ds(0, APT_LAST)], semA).wait()

    cp_ae.wait()
    cp_zero.wait()

    def sweep(lo, hi):
        i = lo
        while i < hi:
            g_n = min(8, hi - i)
            sls = [pl.ds((i + j) * L, L) for j in range(g_n)]
            zs = [z_v[sl] for sl in sls]
            gs = [g_v[sl] for sl in sls]
            vals = [plsc.load_gather(ae_v, [z]) for z in zs]
            for g, v in zip(gs, vals):
                plsc.addupdate_scatter(acc_v, [g], v)
            i += g_n

    sweep(0, APT_LAST // L)

    @pl.when(s < NS - 1)
    def _():
        sweep(APT_LAST // L, APT // L)

    pltpu.sync_copy(acc_v, rows_sh.at[s])
    plsc.subcore_barrier()
    pltpu.sync_copy(rows_sh.at[:, pl.ds(g0, GPT)], cmb_v)

    cp_en.wait()
    cp_na.wait()
    cp_sc.wait()
    cp_sh.wait()
    zero16 = jnp.zeros((L,), jnp.int32)
    scale = plsc.load_gather(sc_v, [zero16])
    shift = plsc.load_gather(sh_v, [zero16])
    for k in range(GPT // L):
        sl = pl.ds(k * L, L)
        acc = cmb_v[0, sl]
        for r in range(1, NS):
            acc = acc + cmb_v[r, sl]
        res_v[sl] = (en_v[sl] * scale
                     + na_v[sl].astype(jnp.float32) * shift + acc)
    pltpu.sync_copy(res_v, out_hbm.at[pl.ds(g0, GPT)])

  return _fused


def kernel(energy, n_atoms, Z, image_idx, scale_by, shift_by, atomic_energies):
    zrow = jnp.zeros((ROW,), jnp.float32)
    return _build()(
        energy, n_atoms.astype(jnp.int32), Z.astype(jnp.int32),
        image_idx.astype(jnp.int32), scale_by.astype(jnp.float32),
        shift_by.astype(jnp.float32), atomic_energies.astype(jnp.float32),
        zrow)


# speedup vs baseline: 1.9727x; 1.0087x over previous
import functools

import jax
import jax.numpy as jnp
from jax import lax
from jax.experimental import pallas as pl
from jax.experimental.pallas import tpu as pltpu
from jax.experimental.pallas import tpu_sc as plsc

NG = 4096
NA = 100000
NE = 119
NS, L = 16, 16
ROW = 4104
GPT = NG // NS
APT = 6272
APT_LAST = NA - (NS - 1) * APT


@functools.cache
def _build():
  mesh = plsc.VectorSubcoreMesh(
      core_axis_name="c", subcore_axis_name="s",
      num_cores=1, num_subcores=NS)

  @functools.partial(
      pl.kernel,
      out_type=jax.ShapeDtypeStruct((NG,), jnp.float32),
      mesh=mesh,
      compiler_params=pltpu.CompilerParams(needs_layout_passes=False),
      scratch_types=[
          pltpu.VMEM((APT,), jnp.int32),
          pltpu.VMEM((APT,), jnp.int32),
          pltpu.VMEM((NE,), jnp.float32),
          pltpu.VMEM((ROW,), jnp.float32),
          pltpu.VMEM((NS, GPT), jnp.float32),
          pltpu.VMEM((GPT,), jnp.float32),
          pltpu.VMEM((GPT,), jnp.int32),
          pltpu.VMEM((1,), jnp.float32),
          pltpu.VMEM((1,), jnp.float32),
          pltpu.VMEM((GPT,), jnp.float32),
          pltpu.VMEM_SHARED((NS, ROW), jnp.float32),
          pltpu.SemaphoreType.DMA,
          pltpu.SemaphoreType.DMA,
      ],
  )
  def _fused(energy_hbm, natoms_hbm, z_hbm, img_hbm, scale_hbm, shift_hbm,
             ae_hbm, zrow_hbm, out_hbm,
             z_v, g_v, ae_v, acc_v, cmb_v, en_v, na_v, sc_v, sh_v, res_v,
             rows_sh, semA, semB):
    s = lax.axis_index("s")
    base = s * APT
    g0 = s * GPT

    cp_ae = pltpu.async_copy(ae_hbm, ae_v, semA)
    cp_zero = pltpu.async_copy(zrow_hbm, acc_v, semA)
    cp_en = pltpu.async_copy(energy_hbm.at[pl.ds(g0, GPT)], en_v, semB)
    cp_na = pltpu.async_copy(natoms_hbm.at[pl.ds(g0, GPT)], na_v, semB)
    cp_sc = pltpu.async_copy(scale_hbm, sc_v, semB)
    cp_sh = pltpu.async_copy(shift_hbm, sh_v, semB)

    @pl.when(s < NS - 1)
    def _():
        pltpu.async_copy(z_hbm.at[pl.ds(base, APT)], z_v, semA).wait()
        pltpu.async_copy(img_hbm.at[pl.ds(base, APT)], g_v, semA).wait()

    @pl.when(s == NS - 1)
    def _():
        pltpu.async_copy(z_hbm.at[pl.ds(base, APT_LAST)],
                         z_v.at[pl.ds(0, APT_LAST)], semA).wait()
        pltpu.async_copy(img_hbm.at[pl.ds(base, APT_LAST)],
                         g_v.at[pl.ds(0, APT_LAST)], semA).wait()

    with jax.named_scope("ph_dma_in"):
        cp_ae.wait()
        cp_zero.wait()

    def sweep(lo, hi):
        i = lo
        while i < hi:
            g_n = min(8, hi - i)
            sls = [pl.ds((i + j) * L, L) for j in range(g_n)]
            zs = [z_v[sl] for sl in sls]
            gs = [g_v[sl] for sl in sls]
            vals = [plsc.load_gather(ae_v, [z]) for z in zs]
            for g, v in zip(gs, vals):
                plsc.addupdate_scatter(acc_v, [g], v)
            i += g_n

    with jax.named_scope("ph_sweep"):
        sweep(0, APT_LAST // L)

        @pl.when(s < NS - 1)
        def _():
            sweep(APT_LAST // L, APT // L)

    with jax.named_scope("ph_stage"):
        pltpu.sync_copy(acc_v, rows_sh.at[s])
        plsc.subcore_barrier()
        pltpu.sync_copy(rows_sh.at[:, pl.ds(g0, GPT)], cmb_v)

    with jax.named_scope("ph_finish"):
        cp_en.wait()
        cp_na.wait()
        cp_sc.wait()
        cp_sh.wait()
        zero16 = jnp.zeros((L,), jnp.int32)
        scale = plsc.load_gather(sc_v, [zero16])
        shift = plsc.load_gather(sh_v, [zero16])
        for k in range(GPT // L):
            sl = pl.ds(k * L, L)
            acc = cmb_v[0, sl]
            for r in range(1, NS):
                acc = acc + cmb_v[r, sl]
            res_v[sl] = (en_v[sl] * scale
                         + na_v[sl].astype(jnp.float32) * shift + acc)
        pltpu.sync_copy(res_v, out_hbm.at[pl.ds(g0, GPT)])

  return _fused


def kernel(energy, n_atoms, Z, image_idx, scale_by, shift_by, atomic_energies):
    zrow = jnp.zeros((ROW,), jnp.float32)
    return _build()(
        energy, n_atoms.astype(jnp.int32), Z.astype(jnp.int32),
        image_idx.astype(jnp.int32), scale_by.astype(jnp.float32),
        shift_by.astype(jnp.float32), atomic_energies.astype(jnp.float32),
        zrow)


# speedup vs baseline: 2.2788x; 1.1552x over previous
import functools

import jax
import jax.numpy as jnp
from jax import lax
from jax.experimental import pallas as pl
from jax.experimental.pallas import tpu as pltpu
from jax.experimental.pallas import tpu_sc as plsc

NG = 4096
NA = 100000
NE = 119
NS, L = 16, 16
ROW = 4104
GPT = NG // NS
APT = 6272
APT_LAST = NA - (NS - 1) * APT


@functools.cache
def _build():
  mesh = plsc.VectorSubcoreMesh(
      core_axis_name="c", subcore_axis_name="s",
      num_cores=1, num_subcores=NS)

  @functools.partial(
      pl.kernel,
      out_type=jax.ShapeDtypeStruct((NG,), jnp.float32),
      mesh=mesh,
      compiler_params=pltpu.CompilerParams(needs_layout_passes=False),
      scratch_types=[
          pltpu.VMEM((APT,), jnp.int32),
          pltpu.VMEM((APT,), jnp.int32),
          pltpu.VMEM((NE,), jnp.float32),
          pltpu.VMEM((ROW,), jnp.float32),
          pltpu.VMEM((NS, GPT), jnp.float32),
          pltpu.VMEM((GPT,), jnp.float32),
          pltpu.VMEM((GPT,), jnp.int32),
          pltpu.VMEM((1,), jnp.float32),
          pltpu.VMEM((1,), jnp.float32),
          pltpu.VMEM((GPT,), jnp.float32),
          pltpu.VMEM_SHARED((NS, ROW), jnp.float32),
          pltpu.SemaphoreType.DMA,
          pltpu.SemaphoreType.DMA,
      ],
  )
  def _fused(energy_hbm, natoms_hbm, z_hbm, img_hbm, scale_hbm, shift_hbm,
             ae_hbm, zrow_hbm, out_hbm,
             z_v, g_v, ae_v, acc_v, cmb_v, en_v, na_v, sc_v, sh_v, res_v,
             rows_sh, semA, semB):
    s = lax.axis_index("s")
    base = jnp.minimum(s * APT, NA - APT)
    g0 = s * GPT

    cp_ae = pltpu.async_copy(ae_hbm, ae_v, semA)
    cp_zero = pltpu.async_copy(zrow_hbm, acc_v, semA)
    cp_en = pltpu.async_copy(energy_hbm.at[pl.ds(g0, GPT)], en_v, semB)
    cp_na = pltpu.async_copy(natoms_hbm.at[pl.ds(g0, GPT)], na_v, semB)
    cp_sc = pltpu.async_copy(scale_hbm, sc_v, semB)
    cp_sh = pltpu.async_copy(shift_hbm, sh_v, semB)

    off = pl.multiple_of(base, 8)
    cp_z = pltpu.async_copy(z_hbm.at[pl.ds(off, APT)], z_v, semA)
    cp_g = pltpu.async_copy(img_hbm.at[pl.ds(off, APT)], g_v, semA)

    with jax.named_scope("ph_dma_in"):
        cp_ae.wait()
        cp_zero.wait()
        cp_z.wait()
        cp_g.wait()

    NV = APT // L
    iota = lax.iota(jnp.int32, L)
    lane_base = iota * NV
    D = NS * APT - NA
    m_pre = jnp.logical_or(iota != 0, jnp.full((L,), s < NS - 1))

    with jax.named_scope("ph_sweep"):
        i = 0
        while i < NV:
            g_n = min(8, NV - i)
            idxs = [lane_base + (i + j) for j in range(g_n)]
            zs = [plsc.load_gather(z_v, [ix]) for ix in idxs]
            gs = [plsc.load_gather(g_v, [ix]) for ix in idxs]
            vals = [plsc.load_gather(ae_v, [z]) for z in zs]
            for j, (g, v) in enumerate(zip(gs, vals)):
                if i + j < D:
                    plsc.addupdate_scatter(acc_v, [g], v, mask=m_pre)
                else:
                    plsc.addupdate_scatter(acc_v, [g], v)
            i += g_n

    with jax.named_scope("ph_stage"):
        pltpu.sync_copy(acc_v, rows_sh.at[s])
        plsc.subcore_barrier()
        pltpu.sync_copy(rows_sh.at[:, pl.ds(g0, GPT)], cmb_v)

    with jax.named_scope("ph_finish"):
        cp_en.wait()
        cp_na.wait()
        cp_sc.wait()
        cp_sh.wait()
        zero16 = jnp.zeros((L,), jnp.int32)
        scale = plsc.load_gather(sc_v, [zero16])
        shift = plsc.load_gather(sh_v, [zero16])
        for k in range(GPT // L):
            sl = pl.ds(k * L, L)
            acc = cmb_v[0, sl]
            for r in range(1, NS):
                acc = acc + cmb_v[r, sl]
            res_v[sl] = (en_v[sl] * scale
                         + na_v[sl].astype(jnp.float32) * shift + acc)
        pltpu.sync_copy(res_v, out_hbm.at[pl.ds(g0, GPT)])

  return _fused


def kernel(energy, n_atoms, Z, image_idx, scale_by, shift_by, atomic_energies):
    zrow = jnp.zeros((ROW,), jnp.float32)
    return _build()(
        energy, n_atoms.astype(jnp.int32), Z.astype(jnp.int32),
        image_idx.astype(jnp.int32), scale_by.astype(jnp.float32),
        shift_by.astype(jnp.float32), atomic_energies.astype(jnp.float32),
        zrow)


# speedup vs baseline: 2.5872x; 1.1353x over previous
import functools

import jax
import jax.numpy as jnp
from jax import lax
from jax.experimental import pallas as pl
from jax.experimental.pallas import tpu as pltpu
from jax.experimental.pallas import tpu_sc as plsc

NG = 4096
NA = 100000
NE = 119
NS, L = 16, 16
ROW = 4104
GPT = NG // NS
APT = 6272
APT_LAST = NA - (NS - 1) * APT


@functools.cache
def _build():
  mesh = plsc.VectorSubcoreMesh(
      core_axis_name="c", subcore_axis_name="s",
      num_cores=1, num_subcores=NS)

  @functools.partial(
      pl.kernel,
      out_type=jax.ShapeDtypeStruct((NG,), jnp.float32),
      mesh=mesh,
      compiler_params=pltpu.CompilerParams(needs_layout_passes=False),
      scratch_types=[
          pltpu.VMEM((APT,), jnp.int32),
          pltpu.VMEM((APT,), jnp.int32),
          pltpu.VMEM((NE,), jnp.float32),
          pltpu.VMEM((ROW,), jnp.float32),
          pltpu.VMEM((NS, GPT), jnp.float32),
          pltpu.VMEM((GPT,), jnp.float32),
          pltpu.VMEM((GPT,), jnp.int32),
          pltpu.VMEM((1,), jnp.float32),
          pltpu.VMEM((1,), jnp.float32),
          pltpu.VMEM((GPT,), jnp.float32),
          pltpu.VMEM_SHARED((NS, ROW), jnp.float32),
          pltpu.SemaphoreType.DMA,
          pltpu.SemaphoreType.DMA,
      ],
  )
  def _fused(energy_hbm, natoms_hbm, z_hbm, img_hbm, scale_hbm, shift_hbm,
             ae_hbm, zrow_hbm, out_hbm,
             z_v, g_v, ae_v, acc_v, cmb_v, en_v, na_v, sc_v, sh_v, res_v,
             rows_sh, semA, semB):
    s = lax.axis_index("s")
    base = jnp.minimum(s * APT, NA - APT)
    g0 = s * GPT

    cp_ae = pltpu.async_copy(ae_hbm, ae_v, semA)
    cp_zero = pltpu.async_copy(zrow_hbm, acc_v, semA)
    cp_en = pltpu.async_copy(energy_hbm.at[pl.ds(g0, GPT)], en_v, semB)
    cp_na = pltpu.async_copy(natoms_hbm.at[pl.ds(g0, GPT)], na_v, semB)
    cp_sc = pltpu.async_copy(scale_hbm, sc_v, semB)
    cp_sh = pltpu.async_copy(shift_hbm, sh_v, semB)

    off = pl.multiple_of(base, 8)
    cp_z = pltpu.async_copy(z_hbm.at[pl.ds(off, APT)], z_v, semA)
    cp_g = pltpu.async_copy(img_hbm.at[pl.ds(off, APT)], g_v, semA)

    with jax.named_scope("ph_dma_in"):
        cp_ae.wait()
        cp_zero.wait()
        cp_z.wait()
        cp_g.wait()

    NV = APT // L
    iota = lax.iota(jnp.int32, L)
    lane_base = iota * NV
    D = NS * APT - NA
    m_pre = jnp.logical_or(iota != 0, jnp.full((L,), s < NS - 1))

    with jax.named_scope("ph_sweep"):
        GRP = 8

        def grp(i0, masked):
            idxs = [lane_base + (i0 + j) for j in range(GRP)]
            zs = [plsc.load_gather(z_v, [ix]) for ix in idxs]
            gs = [plsc.load_gather(g_v, [ix]) for ix in idxs]
            vals = [plsc.load_gather(ae_v, [z]) for z in zs]
            for g, v in zip(gs, vals):
                if masked:
                    plsc.addupdate_scatter(acc_v, [g], v, mask=m_pre)
                else:
                    plsc.addupdate_scatter(acc_v, [g], v)

        def body_masked(k, carry):
            grp(k * GRP, True)
            return carry

        def body_plain(k, carry):
            grp(k * GRP, False)
            return carry

        lax.fori_loop(0, D // GRP, body_masked, 0)
        lax.fori_loop(D // GRP, NV // GRP, body_plain, 0)

    with jax.named_scope("ph_stage"):
        pltpu.sync_copy(acc_v, rows_sh.at[s])
        plsc.subcore_barrier()
        pltpu.sync_copy(rows_sh.at[:, pl.ds(g0, GPT)], cmb_v)

    with jax.named_scope("ph_finish"):
        cp_en.wait()
        cp_na.wait()
        cp_sc.wait()
        cp_sh.wait()
        zero16 = jnp.zeros((L,), jnp.int32)
        scale = plsc.load_gather(sc_v, [zero16])
        shift = plsc.load_gather(sh_v, [zero16])
        for k in range(GPT // L):
            sl = pl.ds(k * L, L)
            acc = cmb_v[0, sl]
            for r in range(1, NS):
                acc = acc + cmb_v[r, sl]
            res_v[sl] = (en_v[sl] * scale
                         + na_v[sl].astype(jnp.float32) * shift + acc)
        pltpu.sync_copy(res_v, out_hbm.at[pl.ds(g0, GPT)])

  return _fused


def kernel(energy, n_atoms, Z, image_idx, scale_by, shift_by, atomic_energies):
    zrow = jnp.zeros((ROW,), jnp.float32)
    return _build()(
        energy, n_atoms.astype(jnp.int32), Z.astype(jnp.int32),
        image_idx.astype(jnp.int32), scale_by.astype(jnp.float32),
        shift_by.astype(jnp.float32), atomic_energies.astype(jnp.float32),
        zrow)


# speedup vs baseline: 2.6179x; 1.0119x over previous
import functools

import jax
import jax.numpy as jnp
from jax import lax
from jax.experimental import pallas as pl
from jax.experimental.pallas import tpu as pltpu
from jax.experimental.pallas import tpu_sc as plsc

NG = 4096
NA = 100000
NE = 119
NS, L = 16, 16
ROW = 4104
GPT = NG // NS
APT = 6272
APT_LAST = NA - (NS - 1) * APT


@functools.cache
def _build():
  mesh = plsc.VectorSubcoreMesh(
      core_axis_name="c", subcore_axis_name="s",
      num_cores=1, num_subcores=NS)

  @functools.partial(
      pl.kernel,
      out_type=jax.ShapeDtypeStruct((NG,), jnp.float32),
      mesh=mesh,
      compiler_params=pltpu.CompilerParams(needs_layout_passes=False),
      scratch_types=[
          pltpu.VMEM((APT,), jnp.int32),
          pltpu.VMEM((APT,), jnp.int32),
          pltpu.VMEM((NE,), jnp.float32),
          pltpu.VMEM((ROW,), jnp.float32),
          pltpu.VMEM((NS, GPT), jnp.float32),
          pltpu.VMEM((GPT,), jnp.float32),
          pltpu.VMEM((GPT,), jnp.int32),
          pltpu.VMEM((1,), jnp.float32),
          pltpu.VMEM((1,), jnp.float32),
          pltpu.VMEM((GPT,), jnp.float32),
          pltpu.VMEM_SHARED((NS, ROW), jnp.float32),
          pltpu.SemaphoreType.DMA,
          pltpu.SemaphoreType.DMA,
      ],
  )
  def _fused(energy_hbm, natoms_hbm, z_hbm, img_hbm, scale_hbm, shift_hbm,
             ae_hbm, zrow_hbm, out_hbm,
             z_v, g_v, ae_v, acc_v, cmb_v, en_v, na_v, sc_v, sh_v, res_v,
             rows_sh, semA, semB):
    s = lax.axis_index("s")
    base = jnp.minimum(s * APT, NA - APT)
    g0 = s * GPT

    cp_ae = pltpu.async_copy(ae_hbm, ae_v, semA)
    cp_zero = pltpu.async_copy(zrow_hbm, acc_v, semA)
    cp_en = pltpu.async_copy(energy_hbm.at[pl.ds(g0, GPT)], en_v, semB)
    cp_na = pltpu.async_copy(natoms_hbm.at[pl.ds(g0, GPT)], na_v, semB)
    cp_sc = pltpu.async_copy(scale_hbm, sc_v, semB)
    cp_sh = pltpu.async_copy(shift_hbm, sh_v, semB)

    off = pl.multiple_of(base, 8)
    cp_z = pltpu.async_copy(z_hbm.at[pl.ds(off, APT)], z_v, semA)
    cp_g = pltpu.async_copy(img_hbm.at[pl.ds(off, APT)], g_v, semA)

    with jax.named_scope("ph_dma_in"):
        cp_ae.wait()
        cp_zero.wait()
        cp_z.wait()
        cp_g.wait()

    NV = APT // L
    iota = lax.iota(jnp.int32, L)
    lane_base = iota * NV
    D = NS * APT - NA

    @pl.when(s == NS - 1)
    def _():
        waste = jnp.full((L,), NG, jnp.int32)
        for k in range(D // L):
            g_v[pl.ds(k * L, L)] = waste

    with jax.named_scope("ph_sweep"):
        GRP = 8

        def body(k, carry):
            i0 = k * GRP
            idxs = [lane_base + (i0 + j) for j in range(GRP)]
            zs = [plsc.load_gather(z_v, [ix]) for ix in idxs]
            gs = [plsc.load_gather(g_v, [ix]) for ix in idxs]
            vals = [plsc.load_gather(ae_v, [z]) for z in zs]
            for g, v in zip(gs, vals):
                plsc.addupdate_scatter(acc_v, [g], v)
            return carry

        lax.fori_loop(0, NV // GRP, body, 0)

    with jax.named_scope("ph_stage"):
        pltpu.sync_copy(acc_v, rows_sh.at[s])
        plsc.subcore_barrier()
        pltpu.sync_copy(rows_sh.at[:, pl.ds(g0, GPT)], cmb_v)

    with jax.named_scope("ph_finish"):
        cp_en.wait()
        cp_na.wait()
        cp_sc.wait()
        cp_sh.wait()
        zero16 = jnp.zeros((L,), jnp.int32)
        scale = plsc.load_gather(sc_v, [zero16])
        shift = plsc.load_gather(sh_v, [zero16])
        for k in range(GPT // L):
            sl = pl.ds(k * L, L)
            acc = cmb_v[0, sl]
            for r in range(1, NS):
                acc = acc + cmb_v[r, sl]
            res_v[sl] = (en_v[sl] * scale
                         + na_v[sl].astype(jnp.float32) * shift + acc)
        pltpu.sync_copy(res_v, out_hbm.at[pl.ds(g0, GPT)])

  return _fused


def kernel(energy, n_atoms, Z, image_idx, scale_by, shift_by, atomic_energies):
    zrow = jnp.zeros((ROW,), jnp.float32)
    return _build()(
        energy, n_atoms.astype(jnp.int32), Z.astype(jnp.int32),
        image_idx.astype(jnp.int32), scale_by.astype(jnp.float32),
        shift_by.astype(jnp.float32), atomic_energies.astype(jnp.float32),
        zrow)
